# Initial kernel scaffold; baseline (speedup 1.0000x reference)
import jax
import jax.numpy as jnp
from jax.experimental import pallas as pl


def kernel(x, edge_index, W1, b1, g1, bt1, W2, b2, g2, bt2, W3, b3):
    raise NotImplementedError("write your pallas kernel here")


# speedup vs baseline: 10.8059x; 10.8059x over previous
import functools
import math

import jax
import jax.numpy as jnp
import numpy as np
from jax import lax
from jax.experimental import pallas as pl
from jax.experimental.pallas import tpu as pltpu
from jax.experimental.pallas import tpu_sc as plsc

NC = 2
NS = 16
NW = NC * NS


def _make_deg_kernel(n, e):
    ept = e // NW
    k = 80
    nb = ept // k
    npt = (n // NS) & ~7
    tail = n - NS * npt
    zr = 104
    assert npt % zr == 0 and zr >= tail

    def body(dst_hbm, ones_hbm, zeros_hbm, out_hbm, dst_idx, ones_v, zeros_v,
             acc):
        c = lax.axis_index("c")
        s = lax.axis_index("s")
        wid = c * NS + s

        pltpu.sync_copy(ones_hbm, ones_v)
        pltpu.sync_copy(zeros_hbm, zeros_v)

        for j in range(npt // zr):
            pltpu.sync_copy(zeros_v, acc.at[pl.ds(s * npt + j * zr, zr)])
        if tail:
            @pl.when(s == NS - 1)
            def _():
                pltpu.sync_copy(zeros_v.at[pl.ds(0, tail)],
                                acc.at[pl.ds(NS * npt, tail)])
        plsc.subcore_barrier()

        def step(b, _):
            off = wid * ept + b * k
            pltpu.sync_copy(dst_hbm.at[pl.ds(off, k)], dst_idx)
            pltpu.sync_copy(ones_v, acc.at[dst_idx], add=True)
            return 0
        lax.fori_loop(0, nb, step, 0)

        plsc.subcore_barrier()
        pltpu.sync_copy(acc.at[pl.ds(s * npt, npt)],
                        out_hbm.at[c, pl.ds(s * npt, npt)])
        if tail:
            @pl.when(s == NS - 1)
            def _():
                pltpu.sync_copy(acc.at[pl.ds(NS * npt, tail)],
                                out_hbm.at[c, pl.ds(NS * npt, tail)])

    return pl.kernel(
        body,
        out_type=jax.ShapeDtypeStruct((NC, n, 128), jnp.float32),
        mesh=plsc.VectorSubcoreMesh(core_axis_name="c", subcore_axis_name="s"),
        scratch_types=[
            pltpu.VMEM((k,), jnp.int32),
            pltpu.VMEM((k, 128), jnp.float32),
            pltpu.VMEM((zr, 128), jnp.float32),
            pltpu.VMEM_SHARED((n, 128), jnp.float32),
        ],
    )


def _make_agg_kernel(n, e, d):
    ept = e // NW
    k = 80
    nb = ept // k
    npt = (n // NS) & ~7
    tail = n - NS * npt
    zr = 104
    assert npt % zr == 0 and zr >= tail

    def body(hp_hbm, src_hbm, dst_hbm, zeros_hbm, out_hbm,
             src_idx, dst_idx, rows, zeros_v, acc, sem):
        c = lax.axis_index("c")
        s = lax.axis_index("s")
        wid = c * NS + s

        pltpu.sync_copy(zeros_hbm, zeros_v)

        for j in range(npt // zr):
            pltpu.sync_copy(zeros_v, acc.at[pl.ds(s * npt + j * zr, zr)])
        if tail:
            @pl.when(s == NS - 1)
            def _():
                pltpu.sync_copy(zeros_v.at[pl.ds(0, tail)],
                                acc.at[pl.ds(NS * npt, tail)])
        plsc.subcore_barrier()

        def step(b, _):
            off = wid * ept + b * k
            pltpu.sync_copy(src_hbm.at[pl.ds(off, k)], src_idx)
            pltpu.sync_copy(dst_hbm.at[pl.ds(off, k)], dst_idx)
            pltpu.async_copy(hp_hbm.at[src_idx], rows, sem).wait()
            pltpu.sync_copy(rows, acc.at[dst_idx], add=True)
            return 0
        lax.fori_loop(0, nb, step, 0)

        plsc.subcore_barrier()
        pltpu.sync_copy(acc.at[pl.ds(s * npt, npt)],
                        out_hbm.at[c, pl.ds(s * npt, npt)])
        if tail:
            @pl.when(s == NS - 1)
            def _():
                pltpu.sync_copy(acc.at[pl.ds(NS * npt, tail)],
                                out_hbm.at[c, pl.ds(NS * npt, tail)])

    return pl.kernel(
        body,
        out_type=jax.ShapeDtypeStruct((NC, n, d), jnp.float32),
        mesh=plsc.VectorSubcoreMesh(core_axis_name="c", subcore_axis_name="s"),
        scratch_types=[
            pltpu.VMEM((k,), jnp.int32),
            pltpu.VMEM((k,), jnp.int32),
            pltpu.VMEM((k, d), jnp.float32),
            pltpu.VMEM((zr, d), jnp.float32),
            pltpu.VMEM_SHARED((n, d), jnp.float32),
            pltpu.SemaphoreType.DMA,
        ],
    )


def _dense_first_body(x_ref, w_ref, degp_ref, hp_ref, dinv_ref):
    deg = degp_ref[0, :, 0:1] + degp_ref[1, :, 0:1] + 1.0
    dinv = lax.rsqrt(deg)
    t = jnp.dot(x_ref[...], w_ref[...], preferred_element_type=jnp.float32)
    hp_ref[...] = t * dinv
    dinv_ref[...] = dinv


def _dense_mid_body(p_ref, hp_ref, dinv_ref, b_ref, g_ref, bt_ref, w_ref,
                    o_ref):
    dinv = dinv_ref[...]
    agg = (p_ref[0] + p_ref[1] + hp_ref[...]) * dinv + b_ref[...]
    scale = g_ref[...] * np.float32(1.0 / math.sqrt(1.0 + 1e-5))
    h = jnp.maximum(agg * scale + bt_ref[...], 0.0)
    o_ref[...] = jnp.dot(h, w_ref[...],
                         preferred_element_type=jnp.float32) * dinv


def _dense_last_body(p_ref, hp_ref, dinv_ref, b_ref, o_ref):
    o_ref[...] = (p_ref[0] + p_ref[1] + hp_ref[...]) * dinv_ref[...] \
        + b_ref[...]


def kernel(x, edge_index, W1, b1, g1, bt1, W2, b2, g2, bt2, W3, b3):
    n, d_in = x.shape
    e = edge_index.shape[1]
    d_h = W1.shape[1]
    d_out = W3.shape[1]
    src = edge_index[0]
    dst = edge_index[1]

    deg_k = _make_deg_kernel(n, e)
    agg_k = _make_agg_kernel(n, e, d_h)
    ones128 = jnp.ones((80, 128), jnp.float32)
    zerosd = jnp.zeros((104, d_h), jnp.float32)

    degp = deg_k(dst, ones128, zerosd)

    f32 = jnp.float32
    hp1, dinv = pl.pallas_call(
        _dense_first_body,
        out_shape=(jax.ShapeDtypeStruct((n, d_h), f32),
                   jax.ShapeDtypeStruct((n, 1), f32)),
    )(x, W1, degp)

    p1 = agg_k(hp1, src, dst, zerosd)

    hp2 = pl.pallas_call(
        _dense_mid_body,
        out_shape=jax.ShapeDtypeStruct((n, d_h), f32),
    )(p1, hp1, dinv, b1.reshape(1, -1), g1.reshape(1, -1),
      bt1.reshape(1, -1), W2)

    p2 = agg_k(hp2, src, dst, zerosd)

    hp3 = pl.pallas_call(
        _dense_mid_body,
        out_shape=jax.ShapeDtypeStruct((n, d_out), f32),
    )(p2, hp2, dinv, b2.reshape(1, -1), g2.reshape(1, -1),
      bt2.reshape(1, -1), W3)

    p3 = agg_k(hp3, src, dst, zerosd)

    out = pl.pallas_call(
        _dense_last_body,
        out_shape=jax.ShapeDtypeStruct((n, d_out), f32),
    )(p3, hp3, dinv, b3.reshape(1, -1))

    return out


# speedup vs baseline: 24.0619x; 2.2267x over previous
import math

import jax
import jax.numpy as jnp
import numpy as np
from jax import lax
from jax.experimental import pallas as pl
from jax.experimental.pallas import tpu as pltpu
from jax.experimental.pallas import tpu_sc as plsc

NC = 2
NS = 16
NW = NC * NS
K = 80


def _zero_acc(acc, zsrc, s, npt, tail):
    for j in range(npt // K):
        pltpu.sync_copy(zsrc, acc.at[pl.ds(s * npt + j * K, K)])
    rem = npt % K
    if rem:
        pltpu.sync_copy(zsrc.at[pl.ds(0, rem)],
                        acc.at[pl.ds(s * npt + (npt // K) * K, rem)])
    if tail:
        @pl.when(s == NS - 1)
        def _():
            pltpu.sync_copy(zsrc.at[pl.ds(0, tail)],
                            acc.at[pl.ds(NS * npt, tail)])


def _write_out(acc, out_hbm, c, s, npt, tail):
    pltpu.sync_copy(acc.at[pl.ds(s * npt, npt)],
                    out_hbm.at[c, pl.ds(s * npt, npt)])
    if tail:
        @pl.when(s == NS - 1)
        def _():
            pltpu.sync_copy(acc.at[pl.ds(NS * npt, tail)],
                            out_hbm.at[c, pl.ds(NS * npt, tail)])


def _make_deg_kernel(n, e):
    ept = e // NW
    nb = ept // K
    npt = (n // NS) & ~7
    tail = n - NS * npt
    bat = 5
    assert nb % bat == 0

    def body(dst3_hbm, ones_hbm, zeros_hbm, out_hbm,
             dst_v, ones_v, zeros_v, acc, sem):
        c = lax.axis_index("c")
        s = lax.axis_index("s")
        wid = c * NS + s

        pltpu.sync_copy(ones_hbm, ones_v)
        pltpu.sync_copy(zeros_hbm, zeros_v)
        pltpu.sync_copy(dst3_hbm.at[wid], dst_v)
        _zero_acc(acc, zeros_v, s, npt, tail)
        plsc.subcore_barrier()

        def step(t, _):
            descs = [
                pltpu.async_copy(ones_v, acc.at[dst_v.at[bat * t + i]],
                                 sem, add=True)
                for i in range(bat)
            ]
            for dsc in descs:
                dsc.wait()
            return 0
        lax.fori_loop(0, nb // bat, step, 0)

        plsc.subcore_barrier()
        _write_out(acc, out_hbm, c, s, npt, tail)

    return pl.kernel(
        body,
        out_type=jax.ShapeDtypeStruct((NC, n, 128), jnp.float32),
        mesh=plsc.VectorSubcoreMesh(core_axis_name="c", subcore_axis_name="s"),
        scratch_types=[
            pltpu.VMEM((nb, K), jnp.int32),
            pltpu.VMEM((K, 128), jnp.float32),
            pltpu.VMEM((K, 128), jnp.float32),
            pltpu.VMEM_SHARED((n, 128), jnp.float32),
            pltpu.SemaphoreType.DMA,
        ],
    )


def _make_agg_kernel(n, e, d):
    assert d == 128
    ept = e // NW
    nb = ept // K
    npt = (n // NS) & ~7
    tail = n - NS * npt
    pairs = nb // 2

    def body(hp_hbm, src_hbm, dst3_hbm, zeros_hbm, out_hbm,
             src_v, dst_v, rows_a, rows_b, acc, sem_a, sem_b):
        c = lax.axis_index("c")
        s = lax.axis_index("s")
        wid = c * NS + s

        pltpu.sync_copy(src_hbm.at[pl.ds(wid * ept, ept)], src_v)
        pltpu.sync_copy(dst3_hbm.at[wid], dst_v)
        pltpu.sync_copy(zeros_hbm, rows_a)
        _zero_acc(acc, rows_a, s, npt, tail)
        plsc.subcore_barrier()

        pltpu.async_copy(hp_hbm.at[src_v.at[pl.ds(0, K)]], rows_a, sem_a)

        def pair(p, _):
            b1 = 2 * p + 1
            b2 = 2 * p + 2
            pltpu.async_copy(hp_hbm.at[src_v.at[pl.ds(b1 * K, K)]],
                             rows_b, sem_b)
            pltpu.make_async_copy(hp_hbm.at[pl.ds(0, K)], rows_a,
                                  sem_a).wait()
            pltpu.sync_copy(rows_a, acc.at[dst_v.at[2 * p]], add=True)

            @pl.when(b2 < nb)
            def _():
                pltpu.async_copy(hp_hbm.at[src_v.at[pl.ds(b2 * K, K)]],
                                 rows_a, sem_a)
            pltpu.make_async_copy(hp_hbm.at[pl.ds(0, K)], rows_b,
                                  sem_b).wait()
            pltpu.sync_copy(rows_b, acc.at[dst_v.at[b1]], add=True)
            return 0
        lax.fori_loop(0, pairs, pair, 0)

        if nb % 2:
            pltpu.make_async_copy(hp_hbm.at[pl.ds(0, K)], rows_a,
                                  sem_a).wait()
            pltpu.sync_copy(rows_a, acc.at[dst_v.at[nb - 1]], add=True)

        plsc.subcore_barrier()
        _write_out(acc, out_hbm, c, s, npt, tail)

    return pl.kernel(
        body,
        out_type=jax.ShapeDtypeStruct((NC, n, d), jnp.float32),
        mesh=plsc.VectorSubcoreMesh(core_axis_name="c", subcore_axis_name="s"),
        scratch_types=[
            pltpu.VMEM((ept,), jnp.int32),
            pltpu.VMEM((nb, K), jnp.int32),
            pltpu.VMEM((K, d), jnp.float32),
            pltpu.VMEM((K, d), jnp.float32),
            pltpu.VMEM_SHARED((n, d), jnp.float32),
            pltpu.SemaphoreType.DMA,
            pltpu.SemaphoreType.DMA,
        ],
    )


def _mm_body(x_ref, w_ref, o_ref):
    o_ref[...] = jnp.dot(x_ref[...], w_ref[...],
                         preferred_element_type=jnp.float32)


def _dinv_body(t_ref, degp_ref, hp_ref, dinv_ref):
    deg = degp_ref[0, :, 0:1] + degp_ref[1, :, 0:1] + 1.0
    dinv = lax.rsqrt(deg)
    hp_ref[...] = t_ref[...] * dinv
    dinv_ref[...] = dinv


def _dense_mid_body(p_ref, hp_ref, dinv_ref, b_ref, g_ref, bt_ref, w_ref,
                    o_ref):
    dinv = dinv_ref[...]
    agg = (p_ref[0] + p_ref[1] + hp_ref[...]) * dinv + b_ref[...]
    scale = g_ref[...] * np.float32(1.0 / math.sqrt(1.0 + 1e-5))
    h = jnp.maximum(agg * scale + bt_ref[...], 0.0)
    o_ref[...] = jnp.dot(h, w_ref[...],
                         preferred_element_type=jnp.float32) * dinv


def _dense_last_body(p_ref, hp_ref, dinv_ref, b_ref, o_ref):
    o_ref[...] = (p_ref[0] + p_ref[1] + hp_ref[...]) * dinv_ref[...] \
        + b_ref[...]


def kernel(x, edge_index, W1, b1, g1, bt1, W2, b2, g2, bt2, W3, b3):
    n, d_in = x.shape
    e = edge_index.shape[1]
    d_h = W1.shape[1]
    d_out = W3.shape[1]
    src = edge_index[0]
    dst = edge_index[1]
    ept = e // NW
    dst3 = dst.reshape(NW, ept // K, K)

    deg_k = _make_deg_kernel(n, e)
    agg_k = _make_agg_kernel(n, e, d_h)
    ones128 = jnp.ones((K, 128), jnp.float32)
    zerosd = jnp.zeros((K, d_h), jnp.float32)

    f32 = jnp.float32
    t1 = pl.pallas_call(
        _mm_body, out_shape=jax.ShapeDtypeStruct((n, d_h), f32))(x, W1)
    degp = deg_k(dst3, ones128, zerosd)
    hp1, dinv = pl.pallas_call(
        _dinv_body,
        out_shape=(jax.ShapeDtypeStruct((n, d_h), f32),
                   jax.ShapeDtypeStruct((n, 1), f32)),
    )(t1, degp)

    p1 = agg_k(hp1, src, dst3, zerosd)

    hp2 = pl.pallas_call(
        _dense_mid_body,
        out_shape=jax.ShapeDtypeStruct((n, d_h), f32),
    )(p1, hp1, dinv, b1.reshape(1, -1), g1.reshape(1, -1),
      bt1.reshape(1, -1), W2)

    p2 = agg_k(hp2, src, dst3, zerosd)

    hp3 = pl.pallas_call(
        _dense_mid_body,
        out_shape=jax.ShapeDtypeStruct((n, d_out), f32),
    )(p2, hp2, dinv, b2.reshape(1, -1), g2.reshape(1, -1),
      bt2.reshape(1, -1), W3)

    p3 = agg_k(hp3, src, dst3, zerosd)

    out = pl.pallas_call(
        _dense_last_body,
        out_shape=jax.ShapeDtypeStruct((n, d_out), f32),
    )(p3, hp3, dinv, b3.reshape(1, -1))

    return out


# speedup vs baseline: 27.0257x; 1.1232x over previous
import math

import jax
import jax.numpy as jnp
import numpy as np
from jax import lax
from jax.experimental import pallas as pl
from jax.experimental.pallas import tpu as pltpu
from jax.experimental.pallas import tpu_sc as plsc

NC = 2
NS = 16
NW = NC * NS
K = 80


def _zero_acc(acc, zsrc, s, npt, tail):
    for j in range(npt // K):
        pltpu.sync_copy(zsrc, acc.at[pl.ds(s * npt + j * K, K)])
    rem = npt % K
    if rem:
        pltpu.sync_copy(zsrc.at[pl.ds(0, rem)],
                        acc.at[pl.ds(s * npt + (npt // K) * K, rem)])
    if tail:
        @pl.when(s == NS - 1)
        def _():
            pltpu.sync_copy(zsrc.at[pl.ds(0, tail)],
                            acc.at[pl.ds(NS * npt, tail)])


def _write_out(acc, out_hbm, c, s, npt, tail):
    pltpu.sync_copy(acc.at[pl.ds(s * npt, npt)],
                    out_hbm.at[c, pl.ds(s * npt, npt)])
    if tail:
        @pl.when(s == NS - 1)
        def _():
            pltpu.sync_copy(acc.at[pl.ds(NS * npt, tail)],
                            out_hbm.at[c, pl.ds(NS * npt, tail)])


def _make_deg_kernel(n, e):
    ept = e // NW
    nb = ept // K
    npt = (n // NS) & ~7
    tail = n - NS * npt
    bat = 5
    assert nb % bat == 0

    def body(dst3_hbm, ones_hbm, zeros_hbm, out_hbm,
             dst_v, ones_v, zeros_v, acc, sem):
        c = lax.axis_index("c")
        s = lax.axis_index("s")
        wid = c * NS + s

        pltpu.sync_copy(ones_hbm, ones_v)
        pltpu.sync_copy(zeros_hbm, zeros_v)
        pltpu.sync_copy(dst3_hbm.at[wid], dst_v)
        _zero_acc(acc, zeros_v, s, npt, tail)
        plsc.subcore_barrier()

        def step(t, _):
            descs = [
                pltpu.async_copy(ones_v, acc.at[dst_v.at[bat * t + i]],
                                 sem, add=True)
                for i in range(bat)
            ]
            for dsc in descs:
                dsc.wait()
            return 0
        lax.fori_loop(0, nb // bat, step, 0)

        plsc.subcore_barrier()
        _write_out(acc, out_hbm, c, s, npt, tail)

    return pl.kernel(
        body,
        out_type=jax.ShapeDtypeStruct((NC, n, 128), jnp.float32),
        mesh=plsc.VectorSubcoreMesh(core_axis_name="c", subcore_axis_name="s"),
        scratch_types=[
            pltpu.VMEM((nb, K), jnp.int32),
            pltpu.VMEM((K, 128), jnp.float32),
            pltpu.VMEM((K, 128), jnp.float32),
            pltpu.VMEM_SHARED((n, 128), jnp.float32),
            pltpu.SemaphoreType.DMA,
        ],
    )


def _make_agg_kernel(n, e, d):
    assert d == 128
    ept = e // NW
    nb = ept // K
    npt = (n // NS) & ~7
    tail = n - NS * npt
    ch = ((nb // 2 + 23) // 24) * 24
    triples = nb // 3
    rest = nb - 3 * triples
    assert ch % 24 == 0 and nb - ch <= ch and rest == 2

    def body(hp_hbm, src_hbm, dst3_hbm, zeros_hbm, out_hbm,
             src_v, dst_v, rows_a, rows_b, rows_c,
             acc, sem_a, sem_b, sem_c):
        c = lax.axis_index("c")
        s = lax.axis_index("s")
        wid = c * NS + s
        rows = (rows_a, rows_b, rows_c)
        sems = (sem_a, sem_b, sem_c)

        pltpu.sync_copy(src_hbm.at[pl.ds(wid * ept, ept)], src_v)
        pltpu.sync_copy(dst3_hbm.at[wid, pl.ds(0, ch)], dst_v)
        pltpu.sync_copy(zeros_hbm, rows_a)
        _zero_acc(acc, rows_a, s, npt, tail)
        plsc.subcore_barrier()

        def gather(b, buf, sem):
            pltpu.async_copy(hp_hbm.at[src_v.at[pl.ds(b * K, K)]],
                             rows[buf], sems[sem])

        def drain(buf, sem):
            pltpu.make_async_copy(hp_hbm.at[pl.ds(0, K)], rows[buf],
                                  sems[sem]).wait()

        def scatter(b, buf):
            j = jnp.where(b >= ch, b - ch, b)
            pltpu.sync_copy(rows[buf], acc.at[dst_v.at[j]], add=True)

        gather(0, 0, 0)
        gather(1, 1, 1)

        def triple(p, _):
            b = 3 * p

            @pl.when(b == ch)
            def _():
                pltpu.sync_copy(dst3_hbm.at[wid, pl.ds(ch, nb - ch)],
                                dst_v.at[pl.ds(0, nb - ch)])
            for i in range(3):
                drain(i, i)
                gather(b + i + 2, (i + 2) % 3, (i + 2) % 3)
                scatter(b + i, i)
            return 0
        lax.fori_loop(0, triples, triple, 0)
        for i in range(rest):
            b = 3 * triples + i
            drain(b % 3, b % 3)
            scatter(b, b % 3)

        plsc.subcore_barrier()
        _write_out(acc, out_hbm, c, s, npt, tail)

    return pl.kernel(
        body,
        out_type=jax.ShapeDtypeStruct((NC, n, d), jnp.float32),
        mesh=plsc.VectorSubcoreMesh(core_axis_name="c", subcore_axis_name="s"),
        scratch_types=[
            pltpu.VMEM((ept,), jnp.int32),
            pltpu.VMEM((ch, K), jnp.int32),
            pltpu.VMEM((K, d), jnp.float32),
            pltpu.VMEM((K, d), jnp.float32),
            pltpu.VMEM((K, d), jnp.float32),
            pltpu.VMEM_SHARED((n, d), jnp.float32),
            pltpu.SemaphoreType.DMA,
            pltpu.SemaphoreType.DMA,
            pltpu.SemaphoreType.DMA,
        ],
    )


def _mm_body(x_ref, w_ref, o_ref):
    o_ref[...] = jnp.dot(x_ref[...], w_ref[...],
                         preferred_element_type=jnp.float32)


def _dinv_body(t_ref, degp_ref, hp_ref, dinv_ref):
    deg = degp_ref[0, :, 0:1] + degp_ref[1, :, 0:1] + 1.0
    dinv = lax.rsqrt(deg)
    hp_ref[...] = t_ref[...] * dinv
    dinv_ref[...] = dinv


def _dense_mid_body(p_ref, hp_ref, dinv_ref, b_ref, g_ref, bt_ref, w_ref,
                    o_ref):
    dinv = dinv_ref[...]
    agg = (p_ref[0] + p_ref[1] + hp_ref[...]) * dinv + b_ref[...]
    scale = g_ref[...] * np.float32(1.0 / math.sqrt(1.0 + 1e-5))
    h = jnp.maximum(agg * scale + bt_ref[...], 0.0)
    o_ref[...] = jnp.dot(h, w_ref[...],
                         preferred_element_type=jnp.float32) * dinv


def _dense_last_body(p_ref, hp_ref, dinv_ref, b_ref, o_ref):
    o_ref[...] = (p_ref[0] + p_ref[1] + hp_ref[...]) * dinv_ref[...] \
        + b_ref[...]


def kernel(x, edge_index, W1, b1, g1, bt1, W2, b2, g2, bt2, W3, b3):
    n, d_in = x.shape
    e = edge_index.shape[1]
    d_h = W1.shape[1]
    d_out = W3.shape[1]
    src = edge_index[0]
    dst = edge_index[1]
    ept = e // NW
    dst3 = dst.reshape(NW, ept // K, K)

    deg_k = _make_deg_kernel(n, e)
    agg_k = _make_agg_kernel(n, e, d_h)
    ones128 = jnp.ones((K, 128), jnp.float32)
    zerosd = jnp.zeros((K, d_h), jnp.float32)

    f32 = jnp.float32
    t1 = pl.pallas_call(
        _mm_body, out_shape=jax.ShapeDtypeStruct((n, d_h), f32))(x, W1)
    degp = deg_k(dst3, ones128, zerosd)
    hp1, dinv = pl.pallas_call(
        _dinv_body,
        out_shape=(jax.ShapeDtypeStruct((n, d_h), f32),
                   jax.ShapeDtypeStruct((n, 1), f32)),
    )(t1, degp)

    p1 = agg_k(hp1, src, dst3, zerosd)

    hp2 = pl.pallas_call(
        _dense_mid_body,
        out_shape=jax.ShapeDtypeStruct((n, d_h), f32),
    )(p1, hp1, dinv, b1.reshape(1, -1), g1.reshape(1, -1),
      bt1.reshape(1, -1), W2)

    p2 = agg_k(hp2, src, dst3, zerosd)

    hp3 = pl.pallas_call(
        _dense_mid_body,
        out_shape=jax.ShapeDtypeStruct((n, d_out), f32),
    )(p2, hp2, dinv, b2.reshape(1, -1), g2.reshape(1, -1),
      bt2.reshape(1, -1), W3)

    p3 = agg_k(hp3, src, dst3, zerosd)

    out = pl.pallas_call(
        _dense_last_body,
        out_shape=jax.ShapeDtypeStruct((n, d_out), f32),
    )(p3, hp3, dinv, b3.reshape(1, -1))

    return out


# speedup vs baseline: 30.5920x; 1.1320x over previous
import math

import jax
import jax.numpy as jnp
import numpy as np
from jax import lax
from jax.experimental import pallas as pl
from jax.experimental.pallas import tpu as pltpu
from jax.experimental.pallas import tpu_sc as plsc

NC = 2
NS = 16
NW = NC * NS
K = 80


def _zero_acc(acc, zsrc, s, npt, tail):
    for j in range(npt // K):
        pltpu.sync_copy(zsrc, acc.at[pl.ds(s * npt + j * K, K)])
    rem = npt % K
    if rem:
        pltpu.sync_copy(zsrc.at[pl.ds(0, rem)],
                        acc.at[pl.ds(s * npt + (npt // K) * K, rem)])
    if tail:
        @pl.when(s == NS - 1)
        def _():
            pltpu.sync_copy(zsrc.at[pl.ds(0, tail)],
                            acc.at[pl.ds(NS * npt, tail)])


def _write_out(acc, out_hbm, c, s, npt, tail):
    pltpu.sync_copy(acc.at[pl.ds(s * npt, npt)],
                    out_hbm.at[c, pl.ds(s * npt, npt)])
    if tail:
        @pl.when(s == NS - 1)
        def _():
            pltpu.sync_copy(acc.at[pl.ds(NS * npt, tail)],
                            out_hbm.at[c, pl.ds(NS * npt, tail)])


def _make_deg_kernel(n, e):
    ept = e // NW
    nb = ept // K
    npad = ((n + NS * 128 - 1) // (NS * 128)) * (NS * 128)
    npr = npad // NS
    assert K % 16 == 0

    def body(dst3_hbm, out_hbm, dst_v, hist, red_v, sum_v, stage):
        c = lax.axis_index("c")
        s = lax.axis_index("s")
        wid = c * NS + s

        pltpu.sync_copy(dst3_hbm.at[wid], dst_v)
        zero = jnp.zeros((16,), jnp.float32)

        def fz(i, _):
            hist[pl.ds(i * 16, 16)] = zero
            return 0
        lax.fori_loop(0, npad // 16, fz, 0)

        one = jnp.ones((16,), jnp.float32)

        def step(j, _):
            for i in range(K // 16):
                iv = dst_v[j, pl.ds(16 * i, 16)]
                plsc.addupdate_scatter(hist, [iv], one)
            return 0
        lax.fori_loop(0, nb, step, 0)

        pltpu.sync_copy(hist, stage.at[s])
        plsc.subcore_barrier()

        for r in range(NS):
            pltpu.sync_copy(stage.at[r, pl.ds(s * npr, npr)], red_v.at[r])

        def red(j, _):
            acc16 = red_v[0, pl.ds(16 * j, 16)]
            for r in range(1, NS):
                acc16 = acc16 + red_v[r, pl.ds(16 * j, 16)]
            sum_v[pl.ds(16 * j, 16)] = acc16
            return 0
        lax.fori_loop(0, npr // 16, red, 0)

        pltpu.sync_copy(sum_v, out_hbm.at[c, pl.ds(s * npr, npr)])

    return pl.kernel(
        body,
        out_type=jax.ShapeDtypeStruct((NC, npad), jnp.float32),
        compiler_params=pltpu.CompilerParams(needs_layout_passes=False),
        mesh=plsc.VectorSubcoreMesh(core_axis_name="c", subcore_axis_name="s"),
        scratch_types=[
            pltpu.VMEM((nb, K), jnp.int32),
            pltpu.VMEM((npad,), jnp.float32),
            pltpu.VMEM((NS, npad // NS), jnp.float32),
            pltpu.VMEM((npad // NS,), jnp.float32),
            pltpu.VMEM_SHARED((NS, npad), jnp.float32),
        ],
    )


def _make_agg_kernel(n, e, d):
    assert d == 128
    ept = e // NW
    nb = ept // K
    npt = (n // NS) & ~7
    tail = n - NS * npt
    ch = ((nb // 2 + 23) // 24) * 24
    triples = nb // 3
    rest = nb - 3 * triples
    assert ch % 24 == 0 and nb - ch <= ch and rest == 2

    def body(hp_hbm, src_hbm, dst3_hbm, zeros_hbm, out_hbm,
             src_v, dst_v, rows_a, rows_b, rows_c,
             acc, sem_a, sem_b, sem_c):
        c = lax.axis_index("c")
        s = lax.axis_index("s")
        wid = c * NS + s
        rows = (rows_a, rows_b, rows_c)
        sems = (sem_a, sem_b, sem_c)

        pltpu.sync_copy(src_hbm.at[pl.ds(wid * ept, ept)], src_v)
        pltpu.sync_copy(dst3_hbm.at[wid, pl.ds(0, ch)], dst_v)
        pltpu.sync_copy(zeros_hbm, rows_a)
        _zero_acc(acc, rows_a, s, npt, tail)
        plsc.subcore_barrier()

        def gather(b, buf, sem):
            pltpu.async_copy(hp_hbm.at[src_v.at[pl.ds(b * K, K)]],
                             rows[buf], sems[sem])

        def drain(buf, sem):
            pltpu.make_async_copy(hp_hbm.at[pl.ds(0, K)], rows[buf],
                                  sems[sem]).wait()

        def scatter(b, buf):
            j = jnp.where(b >= ch, b - ch, b)
            pltpu.sync_copy(rows[buf], acc.at[dst_v.at[j]], add=True)

        gather(0, 0, 0)
        gather(1, 1, 1)

        def triple(p, _):
            b = 3 * p

            @pl.when(b == ch)
            def _():
                pltpu.sync_copy(dst3_hbm.at[wid, pl.ds(ch, nb - ch)],
                                dst_v.at[pl.ds(0, nb - ch)])
            for i in range(3):
                drain(i, i)
                gather(b + i + 2, (i + 2) % 3, (i + 2) % 3)
                scatter(b + i, i)
            return 0
        lax.fori_loop(0, triples, triple, 0)
        for i in range(rest):
            b = 3 * triples + i
            drain(b % 3, b % 3)
            scatter(b, b % 3)

        plsc.subcore_barrier()
        _write_out(acc, out_hbm, c, s, npt, tail)

    return pl.kernel(
        body,
        out_type=jax.ShapeDtypeStruct((NC, n, d), jnp.float32),
        mesh=plsc.VectorSubcoreMesh(core_axis_name="c", subcore_axis_name="s"),
        scratch_types=[
            pltpu.VMEM((ept,), jnp.int32),
            pltpu.VMEM((ch, K), jnp.int32),
            pltpu.VMEM((K, d), jnp.float32),
            pltpu.VMEM((K, d), jnp.float32),
            pltpu.VMEM((K, d), jnp.float32),
            pltpu.VMEM_SHARED((n, d), jnp.float32),
            pltpu.SemaphoreType.DMA,
            pltpu.SemaphoreType.DMA,
            pltpu.SemaphoreType.DMA,
        ],
    )


def _mm_body(x_ref, w_ref, o_ref):
    o_ref[...] = jnp.dot(x_ref[...], w_ref[...],
                         preferred_element_type=jnp.float32)


def _dinv_body(t_ref, degp_ref, hp_ref, dinv_ref):
    deg = degp_ref[0] + degp_ref[1] + 1.0
    dinv = lax.rsqrt(deg)
    hp_ref[...] = t_ref[...] * dinv
    dinv_ref[...] = dinv


def _dense_mid_body(p_ref, hp_ref, dinv_ref, b_ref, g_ref, bt_ref, w_ref,
                    o_ref):
    dinv = dinv_ref[...]
    agg = (p_ref[0] + p_ref[1] + hp_ref[...]) * dinv + b_ref[...]
    scale = g_ref[...] * np.float32(1.0 / math.sqrt(1.0 + 1e-5))
    h = jnp.maximum(agg * scale + bt_ref[...], 0.0)
    o_ref[...] = jnp.dot(h, w_ref[...],
                         preferred_element_type=jnp.float32) * dinv


def _dense_last_body(p_ref, hp_ref, dinv_ref, b_ref, o_ref):
    o_ref[...] = (p_ref[0] + p_ref[1] + hp_ref[...]) * dinv_ref[...] \
        + b_ref[...]


def kernel(x, edge_index, W1, b1, g1, bt1, W2, b2, g2, bt2, W3, b3):
    n, d_in = x.shape
    e = edge_index.shape[1]
    d_h = W1.shape[1]
    d_out = W3.shape[1]
    src = edge_index[0]
    dst = edge_index[1]
    ept = e // NW
    dst3 = dst.reshape(NW, ept // K, K)

    deg_k = _make_deg_kernel(n, e)
    agg_k = _make_agg_kernel(n, e, d_h)
    zerosd = jnp.zeros((K, d_h), jnp.float32)

    f32 = jnp.float32
    t1 = pl.pallas_call(
        _mm_body, out_shape=jax.ShapeDtypeStruct((n, d_h), f32))(x, W1)
    degp = deg_k(dst3)
    hp1, dinv = pl.pallas_call(
        _dinv_body,
        out_shape=(jax.ShapeDtypeStruct((n, d_h), f32),
                   jax.ShapeDtypeStruct((n, 1), f32)),
    )(t1, degp[:, :n, None])

    p1 = agg_k(hp1, src, dst3, zerosd)

    hp2 = pl.pallas_call(
        _dense_mid_body,
        out_shape=jax.ShapeDtypeStruct((n, d_h), f32),
    )(p1, hp1, dinv, b1.reshape(1, -1), g1.reshape(1, -1),
      bt1.reshape(1, -1), W2)

    p2 = agg_k(hp2, src, dst3, zerosd)

    hp3 = pl.pallas_call(
        _dense_mid_body,
        out_shape=jax.ShapeDtypeStruct((n, d_out), f32),
    )(p2, hp2, dinv, b2.reshape(1, -1), g2.reshape(1, -1),
      bt2.reshape(1, -1), W3)

    p3 = agg_k(hp3, src, dst3, zerosd)

    out = pl.pallas_call(
        _dense_last_body,
        out_shape=jax.ShapeDtypeStruct((n, d_out), f32),
    )(p3, hp3, dinv, b3.reshape(1, -1))

    return out


# speedup vs baseline: 31.0693x; 1.0156x over previous
import math

import jax
import jax.numpy as jnp
import numpy as np
from jax import lax
from jax.experimental import pallas as pl
from jax.experimental.pallas import tpu as pltpu
from jax.experimental.pallas import tpu_sc as plsc

NC = 2
NS = 16
NW = NC * NS
K = 80


def _zero_acc(acc, zsrc, s, npt, tail):
    for j in range(npt // K):
        pltpu.sync_copy(zsrc, acc.at[pl.ds(s * npt + j * K, K)])
    rem = npt % K
    if rem:
        pltpu.sync_copy(zsrc.at[pl.ds(0, rem)],
                        acc.at[pl.ds(s * npt + (npt // K) * K, rem)])
    if tail:
        @pl.when(s == NS - 1)
        def _():
            pltpu.sync_copy(zsrc.at[pl.ds(0, tail)],
                            acc.at[pl.ds(NS * npt, tail)])


def _write_out(acc, out_hbm, c, s, npt, tail):
    pltpu.sync_copy(acc.at[pl.ds(s * npt, npt)],
                    out_hbm.at[c, pl.ds(s * npt, npt)])
    if tail:
        @pl.when(s == NS - 1)
        def _():
            pltpu.sync_copy(acc.at[pl.ds(NS * npt, tail)],
                            out_hbm.at[c, pl.ds(NS * npt, tail)])


def _make_deg_kernel(n, e):
    ept = e // NW
    nb = ept // K
    npad = ((n + NS * 128 - 1) // (NS * 128)) * (NS * 128)
    npr = npad // NS
    assert K % 16 == 0

    def body(dst3_hbm, out_hbm, dst_v, hist, red_v, sum_v, stage):
        c = lax.axis_index("c")
        s = lax.axis_index("s")
        wid = c * NS + s

        pltpu.sync_copy(dst3_hbm.at[wid], dst_v)
        zero = jnp.zeros((16,), jnp.float32)

        def fz(i, _):
            hist[pl.ds(i * 16, 16)] = zero
            return 0
        lax.fori_loop(0, npad // 16, fz, 0)

        one = jnp.ones((16,), jnp.float32)

        def step(j, _):
            for i in range(K // 16):
                iv = dst_v[j, pl.ds(16 * i, 16)]
                plsc.addupdate_scatter(hist, [iv], one)
            return 0
        lax.fori_loop(0, nb, step, 0)

        pltpu.sync_copy(hist, stage.at[s])
        plsc.subcore_barrier()

        for r in range(NS):
            pltpu.sync_copy(stage.at[r, pl.ds(s * npr, npr)], red_v.at[r])

        def red(j, _):
            acc16 = red_v[0, pl.ds(16 * j, 16)]
            for r in range(1, NS):
                acc16 = acc16 + red_v[r, pl.ds(16 * j, 16)]
            sum_v[pl.ds(16 * j, 16)] = acc16
            return 0
        lax.fori_loop(0, npr // 16, red, 0)

        pltpu.sync_copy(sum_v, out_hbm.at[c, pl.ds(s * npr, npr)])

    return pl.kernel(
        body,
        out_type=jax.ShapeDtypeStruct((NC, npad), jnp.float32),
        compiler_params=pltpu.CompilerParams(needs_layout_passes=False),
        mesh=plsc.VectorSubcoreMesh(core_axis_name="c", subcore_axis_name="s"),
        scratch_types=[
            pltpu.VMEM((nb, K), jnp.int32),
            pltpu.VMEM((npad,), jnp.float32),
            pltpu.VMEM((NS, npad // NS), jnp.float32),
            pltpu.VMEM((npad // NS,), jnp.float32),
            pltpu.VMEM_SHARED((NS, npad), jnp.float32),
        ],
    )


def _make_agg_kernel(n, e, d):
    assert d == 128
    ept = e // NW
    nb = ept // K
    npt = (n // NS) & ~7
    tail = n - NS * npt
    ch = ((nb // 2 + 23) // 24) * 24
    triples = nb // 3
    rest = nb - 3 * triples
    assert ch % 24 == 0 and nb - ch <= ch and rest == 2

    def body(hp_hbm, src_hbm, dst3_hbm, zeros_hbm, out_hbm,
             src_v, dst_v, rows_a, rows_b, rows_c,
             acc, sem_a, sem_b, sem_c):
        c = lax.axis_index("c")
        s = lax.axis_index("s")
        wid = c * NS + s
        rows = (rows_a, rows_b, rows_c)
        sems = (sem_a, sem_b, sem_c)

        d1 = pltpu.async_copy(src_hbm.at[pl.ds(wid * ept, ept)], src_v,
                              sem_a)
        d2 = pltpu.async_copy(dst3_hbm.at[wid, pl.ds(0, ch)], dst_v, sem_b)
        pltpu.sync_copy(zeros_hbm.at[pl.ds(s * npt, npt)],
                        acc.at[pl.ds(s * npt, npt)])
        if tail:
            @pl.when(s == NS - 1)
            def _():
                pltpu.sync_copy(zeros_hbm.at[pl.ds(NS * npt, tail)],
                                acc.at[pl.ds(NS * npt, tail)])
        d1.wait()
        d2.wait()
        plsc.subcore_barrier()

        def gather(b, buf, sem):
            pltpu.async_copy(hp_hbm.at[src_v.at[pl.ds(b * K, K)]],
                             rows[buf], sems[sem])

        def drain(buf, sem):
            pltpu.make_async_copy(hp_hbm.at[pl.ds(0, K)], rows[buf],
                                  sems[sem]).wait()

        def scatter(b, buf):
            j = jnp.where(b >= ch, b - ch, b)
            pltpu.sync_copy(rows[buf], acc.at[dst_v.at[j]], add=True)

        gather(0, 0, 0)
        gather(1, 1, 1)

        def triple(p, _):
            b = 3 * p

            @pl.when(b == ch)
            def _():
                pltpu.sync_copy(dst3_hbm.at[wid, pl.ds(ch, nb - ch)],
                                dst_v.at[pl.ds(0, nb - ch)])
            for i in range(3):
                drain(i, i)
                gather(b + i + 2, (i + 2) % 3, (i + 2) % 3)
                scatter(b + i, i)
            return 0
        lax.fori_loop(0, triples, triple, 0)
        for i in range(rest):
            b = 3 * triples + i
            drain(b % 3, b % 3)
            scatter(b, b % 3)

        plsc.subcore_barrier()
        _write_out(acc, out_hbm, c, s, npt, tail)

    return pl.kernel(
        body,
        out_type=jax.ShapeDtypeStruct((NC, n, d), jnp.float32),
        mesh=plsc.VectorSubcoreMesh(core_axis_name="c", subcore_axis_name="s"),
        scratch_types=[
            pltpu.VMEM((ept,), jnp.int32),
            pltpu.VMEM((ch, K), jnp.int32),
            pltpu.VMEM((K, d), jnp.float32),
            pltpu.VMEM((K, d), jnp.float32),
            pltpu.VMEM((K, d), jnp.float32),
            pltpu.VMEM_SHARED((n, d), jnp.float32),
            pltpu.SemaphoreType.DMA,
            pltpu.SemaphoreType.DMA,
            pltpu.SemaphoreType.DMA,
        ],
    )


def _dense_first_body(x_ref, w_ref, degp_ref, hp_ref, dinv_ref):
    deg = degp_ref[0] + degp_ref[1] + 1.0
    dinv = lax.rsqrt(deg)
    t = jnp.dot(x_ref[...], w_ref[...], preferred_element_type=jnp.float32)
    hp_ref[...] = t * dinv
    dinv_ref[...] = dinv


def _dense_mid_body(p_ref, hp_ref, dinv_ref, b_ref, g_ref, bt_ref, w_ref,
                    o_ref):
    dinv = dinv_ref[...]
    agg = (p_ref[0] + p_ref[1] + hp_ref[...]) * dinv + b_ref[...]
    scale = g_ref[...] * np.float32(1.0 / math.sqrt(1.0 + 1e-5))
    h = jnp.maximum(agg * scale + bt_ref[...], 0.0)
    o_ref[...] = jnp.dot(h, w_ref[...],
                         preferred_element_type=jnp.float32) * dinv


def _dense_last_body(p_ref, hp_ref, dinv_ref, b_ref, o_ref):
    o_ref[...] = (p_ref[0] + p_ref[1] + hp_ref[...]) * dinv_ref[...] \
        + b_ref[...]


def kernel(x, edge_index, W1, b1, g1, bt1, W2, b2, g2, bt2, W3, b3):
    n, d_in = x.shape
    e = edge_index.shape[1]
    d_h = W1.shape[1]
    d_out = W3.shape[1]
    src = edge_index[0]
    dst = edge_index[1]
    ept = e // NW
    dst3 = dst.reshape(NW, ept // K, K)

    deg_k = _make_deg_kernel(n, e)
    agg_k = _make_agg_kernel(n, e, d_h)
    zerosd = jnp.zeros((n, d_h), jnp.float32)

    f32 = jnp.float32
    degp = deg_k(dst3)
    hp1, dinv = pl.pallas_call(
        _dense_first_body,
        out_shape=(jax.ShapeDtypeStruct((n, d_h), f32),
                   jax.ShapeDtypeStruct((n, 1), f32)),
    )(x, W1, degp[:, :n, None])

    p1 = agg_k(hp1, src, dst3, zerosd)

    hp2 = pl.pallas_call(
        _dense_mid_body,
        out_shape=jax.ShapeDtypeStruct((n, d_h), f32),
    )(p1, hp1, dinv, b1.reshape(1, -1), g1.reshape(1, -1),
      bt1.reshape(1, -1), W2)

    p2 = agg_k(hp2, src, dst3, zerosd)

    hp3 = pl.pallas_call(
        _dense_mid_body,
        out_shape=jax.ShapeDtypeStruct((n, d_out), f32),
    )(p2, hp2, dinv, b2.reshape(1, -1), g2.reshape(1, -1),
      bt2.reshape(1, -1), W3)

    p3 = agg_k(hp3, src, dst3, zerosd)

    out = pl.pallas_call(
        _dense_last_body,
        out_shape=jax.ShapeDtypeStruct((n, d_out), f32),
    )(p3, hp3, dinv, b3.reshape(1, -1))

    return out
